# Initial kernel scaffold; baseline (speedup 1.0000x reference)
import jax
import jax.numpy as jnp
from jax.experimental import pallas as pl


def kernel(x, edge_index, W1, b1, g1, be1, W2, b2, g2, be2, W3, b3, g3, be3, W4, b4, g4, be4):
    raise NotImplementedError("write your pallas kernel here")


# speedup vs baseline: 19.9143x; 19.9143x over previous
import functools

import jax
import jax.numpy as jnp
from jax import lax
from jax.experimental import pallas as pl
from jax.experimental.pallas import tpu as pltpu
from jax.experimental.pallas import tpu_sc as plsc

N = 10000
N_PAD = 10240
E = 320000
E_PAD = 327680
NC = 2
NS = 16
NW = NC * NS
ROWS_PER_TILE = N_PAD // NS
EDGES_PER_WORKER = E_PAD // NW


def _mesh():
    return plsc.VectorSubcoreMesh(
        core_axis_name="c", subcore_axis_name="s", num_cores=NC, num_subcores=NS
    )


def _make_deg_kernel():
    CE = 512
    G = CE // 128
    CHUNKS = EDGES_PER_WORKER // CE

    @functools.partial(
        pl.kernel,
        out_type=jax.ShapeDtypeStruct((NC, N_PAD), jnp.float32),
        mesh=_mesh(),
        scratch_types=[
            pltpu.VMEM((G, 128), jnp.int32),
            pltpu.VMEM((128,), jnp.float32),
            pltpu.VMEM_SHARED((N_PAD,), jnp.float32),
        ],
    )
    def deg_kernel(dst2_hbm, zeros_hbm, ones_hbm, out_hbm, dstv, onesv, acc):
        cid = lax.axis_index("c")
        sid = lax.axis_index("s")
        w = sid * NC + cid
        pltpu.sync_copy(
            zeros_hbm.at[pl.ds(sid * ROWS_PER_TILE, ROWS_PER_TILE)],
            acc.at[pl.ds(sid * ROWS_PER_TILE, ROWS_PER_TILE)],
        )
        pltpu.sync_copy(ones_hbm, onesv)
        plsc.subcore_barrier()

        row_base = w * (EDGES_PER_WORKER // 128)

        def body(g, carry):
            r0 = row_base + g * G
            pltpu.sync_copy(dst2_hbm.at[pl.ds(r0, G)], dstv)
            for j in range(G):
                pltpu.sync_copy(onesv, acc.at[dstv.at[j]], add=True)
            return carry

        lax.fori_loop(0, CHUNKS, body, 0)
        plsc.subcore_barrier()
        pltpu.sync_copy(
            acc.at[pl.ds(sid * ROWS_PER_TILE, ROWS_PER_TILE)],
            out_hbm.at[cid, pl.ds(sid * ROWS_PER_TILE, ROWS_PER_TILE)],
        )

    return deg_kernel


def _make_agg_kernel(D, CE):
    G = CE // 128
    CHUNKS = EDGES_PER_WORKER // CE

    @functools.partial(
        pl.kernel,
        out_type=jax.ShapeDtypeStruct((NC, N_PAD, D), jnp.float32),
        mesh=_mesh(),
        scratch_types=[
            pltpu.VMEM((CE,), jnp.int32),
            pltpu.VMEM((G, 128), jnp.int32),
            pltpu.VMEM((CE, D), jnp.float32),
            pltpu.VMEM_SHARED((N_PAD, D), jnp.float32),
            pltpu.SemaphoreType.DMA,
        ],
        compiler_params=pltpu.CompilerParams(use_tc_tiling_on_sc=False),
    )
    def agg_kernel(u_hbm, src_hbm, dst2_hbm, zeros_hbm, out_hbm,
                   srcv, dstv, rowsv, acc, sem):
        cid = lax.axis_index("c")
        sid = lax.axis_index("s")
        w = sid * NC + cid
        pltpu.sync_copy(
            zeros_hbm.at[pl.ds(sid * ROWS_PER_TILE, ROWS_PER_TILE)],
            acc.at[pl.ds(sid * ROWS_PER_TILE, ROWS_PER_TILE)],
        )
        plsc.subcore_barrier()

        base = w * EDGES_PER_WORKER
        row_base = w * (EDGES_PER_WORKER // 128)

        def body(g, carry):
            off = base + g * CE
            pltpu.sync_copy(src_hbm.at[pl.ds(off, CE)], srcv)
            pltpu.sync_copy(dst2_hbm.at[pl.ds(row_base + g * G, G)], dstv)
            pltpu.async_copy(u_hbm.at[srcv], rowsv, sem).wait()
            for j in range(G):
                pltpu.sync_copy(
                    rowsv.at[pl.ds(j * 128, 128)],
                    acc.at[dstv.at[j]],
                    add=True,
                )
            return carry

        lax.fori_loop(0, CHUNKS, body, 0)
        plsc.subcore_barrier()
        pltpu.sync_copy(
            acc.at[pl.ds(sid * ROWS_PER_TILE, ROWS_PER_TILE)],
            out_hbm.at[cid, pl.ds(sid * ROWS_PER_TILE, ROWS_PER_TILE)],
        )

    return agg_kernel


def _front_body(deg2t_ref, x_ref, w_ref, u_ref, dinv_ref):
    deg = deg2t_ref[:, 0:1] + deg2t_ref[:, 1:2] + 1.0
    dinv = lax.rsqrt(jnp.maximum(deg, 1.0))
    dinv_ref[...] = dinv
    u_ref[...] = jnp.dot(x_ref[...], w_ref[...],
                         preferred_element_type=jnp.float32) * dinv


def _tc_front(deg2t, x_pad, W1):
    return pl.pallas_call(
        _front_body,
        out_shape=(
            jax.ShapeDtypeStruct((N_PAD, W1.shape[1]), jnp.float32),
            jax.ShapeDtypeStruct((N_PAD, 1), jnp.float32),
        ),
    )(deg2t, x_pad, W1)


def _post_block(acc_ref, u_ref, dinv_ref, b_ref, g_ref, be_ref, mask_ref):
    dinv = dinv_ref[...]
    mask = mask_ref[...]
    conv = (acc_ref[0] + acc_ref[1] + u_ref[...]) * dinv + b_ref[...]
    t = jnp.maximum(conv, 0.0) * mask
    m = jnp.sum(t, axis=0, keepdims=True) * (1.0 / N)
    d = (t - m) * mask
    v = jnp.sum(d * d, axis=0, keepdims=True) * (1.0 / N)
    z = (g_ref[...] * d * lax.rsqrt(v + 1e-5) + be_ref[...]) * mask
    return z, dinv


def _mid_body(acc_ref, u_ref, dinv_ref, b_ref, g_ref, be_ref, mask_ref,
              w_ref, out_ref):
    z, dinv = _post_block(acc_ref, u_ref, dinv_ref, b_ref, g_ref, be_ref,
                          mask_ref)
    out_ref[...] = jnp.dot(z, w_ref[...],
                           preferred_element_type=jnp.float32) * dinv


def _tc_mid(acc, u, dinv, b, g, be, mask, Wn):
    return pl.pallas_call(
        _mid_body,
        out_shape=jax.ShapeDtypeStruct((N_PAD, Wn.shape[1]), jnp.float32),
    )(acc, u, dinv, b, g, be, mask, Wn)


def _back_body(acc_ref, u_ref, dinv_ref, b_ref, g_ref, be_ref, mask_ref,
               out_ref):
    z, _ = _post_block(acc_ref, u_ref, dinv_ref, b_ref, g_ref, be_ref,
                       mask_ref)
    out_ref[...] = z


def _tc_back(acc, u, dinv, b, g, be, mask):
    return pl.pallas_call(
        _back_body,
        out_shape=jax.ShapeDtypeStruct((N_PAD, u.shape[1]), jnp.float32),
    )(acc, u, dinv, b, g, be, mask)


def kernel(x, edge_index, W1, b1, g1, be1, W2, b2, g2, be2,
           W3, b3, g3, be3, W4, b4, g4, be4):
    f32 = jnp.float32
    src = edge_index[0]
    dst = edge_index[1]

    npad = E_PAD - E
    pad_idx = (N + (jnp.arange(npad, dtype=jnp.int32) % (N_PAD - N)))
    src_p = jnp.concatenate([src, pad_idx])
    dst_p = jnp.concatenate([dst, pad_idx])
    dst2 = dst_p.reshape(E_PAD // 128, 128)

    x_pad = jnp.zeros((N_PAD, x.shape[1]), f32).at[:N].set(x)
    mask = (jnp.arange(N_PAD) < N).astype(f32).reshape(N_PAD, 1)
    zeros128 = jnp.zeros((N_PAD, 128), f32)
    zeros64 = jnp.zeros((N_PAD, 64), f32)
    zeros32 = jnp.zeros((N_PAD, 32), f32)
    zeros1 = jnp.zeros((N_PAD,), f32)
    ones128 = jnp.ones((128,), f32)

    deg2 = _make_deg_kernel()(dst2, zeros1, ones128)
    deg2t = deg2.T

    agg128 = _make_agg_kernel(128, 256)
    agg64 = _make_agg_kernel(64, 512)
    agg32 = _make_agg_kernel(32, 1024)

    u1, dinv = _tc_front(deg2t, x_pad, W1)
    acc = agg128(u1, src_p, dst2, zeros128)
    u2 = _tc_mid(acc, u1, dinv, b1.reshape(1, -1), g1.reshape(1, -1),
                 be1.reshape(1, -1), mask, W2)
    acc = agg128(u2, src_p, dst2, zeros128)
    u3 = _tc_mid(acc, u2, dinv, b2.reshape(1, -1), g2.reshape(1, -1),
                 be2.reshape(1, -1), mask, W3)
    acc = agg64(u3, src_p, dst2, zeros64)
    u4 = _tc_mid(acc, u3, dinv, b3.reshape(1, -1), g3.reshape(1, -1),
                 be3.reshape(1, -1), mask, W4)
    acc = agg32(u4, src_p, dst2, zeros32)
    z4 = _tc_back(acc, u4, dinv, b4.reshape(1, -1), g4.reshape(1, -1),
                  be4.reshape(1, -1), mask)
    return z4[:N]


# speedup vs baseline: 24.4317x; 1.2268x over previous
import functools

import jax
import jax.numpy as jnp
from jax import lax
from jax.experimental import pallas as pl
from jax.experimental.pallas import tpu as pltpu
from jax.experimental.pallas import tpu_sc as plsc

N = 10000
N_PAD = 10240
E = 320000
E_PAD = 327680
NC = 2
NS = 16
NW = NC * NS
ROWS_PER_TILE = N_PAD // NS
EDGES_PER_WORKER = E_PAD // NW


def _mesh():
    return plsc.VectorSubcoreMesh(
        core_axis_name="c", subcore_axis_name="s", num_cores=NC, num_subcores=NS
    )


def _make_deg_kernel():
    CE = 512
    G = CE // 128
    CHUNKS = EDGES_PER_WORKER // CE

    @functools.partial(
        pl.kernel,
        out_type=jax.ShapeDtypeStruct((NC, N_PAD), jnp.float32),
        mesh=_mesh(),
        scratch_types=[
            pltpu.VMEM((G, 128), jnp.int32),
            pltpu.VMEM((128,), jnp.float32),
            pltpu.VMEM_SHARED((N_PAD,), jnp.float32),
        ],
    )
    def deg_kernel(dst2_hbm, zeros_hbm, ones_hbm, out_hbm, dstv, onesv, acc):
        cid = lax.axis_index("c")
        sid = lax.axis_index("s")
        w = sid * NC + cid
        pltpu.sync_copy(
            zeros_hbm.at[pl.ds(sid * ROWS_PER_TILE, ROWS_PER_TILE)],
            acc.at[pl.ds(sid * ROWS_PER_TILE, ROWS_PER_TILE)],
        )
        pltpu.sync_copy(ones_hbm, onesv)
        plsc.subcore_barrier()

        row_base = w * (EDGES_PER_WORKER // 128)

        def body(g, carry):
            r0 = row_base + g * G
            pltpu.sync_copy(dst2_hbm.at[pl.ds(r0, G)], dstv)
            for j in range(G):
                pltpu.sync_copy(onesv, acc.at[dstv.at[j]], add=True)
            return carry

        lax.fori_loop(0, CHUNKS, body, 0)
        plsc.subcore_barrier()
        pltpu.sync_copy(
            acc.at[pl.ds(sid * ROWS_PER_TILE, ROWS_PER_TILE)],
            out_hbm.at[cid, pl.ds(sid * ROWS_PER_TILE, ROWS_PER_TILE)],
        )

    return deg_kernel


def _make_agg_kernel(D, CE):
    G = CE // 128
    CHUNKS = EDGES_PER_WORKER // CE

    @functools.partial(
        pl.kernel,
        out_type=jax.ShapeDtypeStruct((NC, N_PAD, D), jnp.float32),
        mesh=_mesh(),
        scratch_types=[
            pltpu.VMEM((CE,), jnp.int32),
            pltpu.VMEM((CE,), jnp.int32),
            pltpu.VMEM((G, 128), jnp.int32),
            pltpu.VMEM((G, 128), jnp.int32),
            pltpu.VMEM((CE, D), jnp.float32),
            pltpu.VMEM((CE, D), jnp.float32),
            pltpu.VMEM_SHARED((N_PAD, D), jnp.float32),
            pltpu.SemaphoreType.DMA,
            pltpu.SemaphoreType.DMA,
        ],
        compiler_params=pltpu.CompilerParams(use_tc_tiling_on_sc=False),
    )
    def agg_kernel(u_hbm, src_hbm, dst2_hbm, zeros_hbm, out_hbm,
                   srcA, srcB, dstA, dstB, rowsA, rowsB, acc, semA, semB):
        cid = lax.axis_index("c")
        sid = lax.axis_index("s")
        w = sid * NC + cid
        pltpu.sync_copy(
            zeros_hbm.at[pl.ds(sid * ROWS_PER_TILE, ROWS_PER_TILE)],
            acc.at[pl.ds(sid * ROWS_PER_TILE, ROWS_PER_TILE)],
        )
        plsc.subcore_barrier()

        base = w * EDGES_PER_WORKER
        row_base = w * (EDGES_PER_WORKER // 128)

        def load_idx(g, srcv, dstv):
            pltpu.sync_copy(src_hbm.at[pl.ds(base + g * CE, CE)], srcv)
            pltpu.sync_copy(dst2_hbm.at[pl.ds(row_base + g * G, G)], dstv)

        def scatter(rowsv, dstv):
            for j in range(G):
                pltpu.sync_copy(
                    rowsv.at[pl.ds(j * 128, 128)],
                    acc.at[dstv.at[j]],
                    add=True,
                )

        load_idx(0, srcA, dstA)
        pltpu.async_copy(u_hbm.at[srcA], rowsA, semA)

        def body(i, carry):
            g0 = 2 * i
            load_idx(g0 + 1, srcB, dstB)
            pltpu.async_copy(u_hbm.at[srcB], rowsB, semB)
            pltpu.make_async_copy(u_hbm.at[srcA], rowsA, semA).wait()
            scatter(rowsA, dstA)

            @pl.when(g0 + 2 < CHUNKS)
            def _():
                load_idx(g0 + 2, srcA, dstA)
                pltpu.async_copy(u_hbm.at[srcA], rowsA, semA)

            pltpu.make_async_copy(u_hbm.at[srcB], rowsB, semB).wait()
            scatter(rowsB, dstB)
            return carry

        lax.fori_loop(0, CHUNKS // 2, body, 0)
        plsc.subcore_barrier()
        pltpu.sync_copy(
            acc.at[pl.ds(sid * ROWS_PER_TILE, ROWS_PER_TILE)],
            out_hbm.at[cid, pl.ds(sid * ROWS_PER_TILE, ROWS_PER_TILE)],
        )

    return agg_kernel


def _front_body(deg2t_ref, x_ref, w_ref, u_ref, dinv_ref):
    deg = deg2t_ref[:, 0:1] + deg2t_ref[:, 1:2] + 1.0
    dinv = lax.rsqrt(jnp.maximum(deg, 1.0))
    dinv_ref[...] = dinv
    u_ref[...] = jnp.dot(x_ref[...], w_ref[...],
                         preferred_element_type=jnp.float32) * dinv


def _tc_front(deg2t, x_pad, W1):
    return pl.pallas_call(
        _front_body,
        out_shape=(
            jax.ShapeDtypeStruct((N_PAD, W1.shape[1]), jnp.float32),
            jax.ShapeDtypeStruct((N_PAD, 1), jnp.float32),
        ),
    )(deg2t, x_pad, W1)


def _post_block(acc_ref, u_ref, dinv_ref, b_ref, g_ref, be_ref, mask_ref):
    dinv = dinv_ref[...]
    mask = mask_ref[...]
    conv = (acc_ref[0] + acc_ref[1] + u_ref[...]) * dinv + b_ref[...]
    t = jnp.maximum(conv, 0.0) * mask
    m = jnp.sum(t, axis=0, keepdims=True) * (1.0 / N)
    d = (t - m) * mask
    v = jnp.sum(d * d, axis=0, keepdims=True) * (1.0 / N)
    z = (g_ref[...] * d * lax.rsqrt(v + 1e-5) + be_ref[...]) * mask
    return z, dinv


def _mid_body(acc_ref, u_ref, dinv_ref, b_ref, g_ref, be_ref, mask_ref,
              w_ref, out_ref):
    z, dinv = _post_block(acc_ref, u_ref, dinv_ref, b_ref, g_ref, be_ref,
                          mask_ref)
    out_ref[...] = jnp.dot(z, w_ref[...],
                           preferred_element_type=jnp.float32) * dinv


def _tc_mid(acc, u, dinv, b, g, be, mask, Wn):
    return pl.pallas_call(
        _mid_body,
        out_shape=jax.ShapeDtypeStruct((N_PAD, Wn.shape[1]), jnp.float32),
    )(acc, u, dinv, b, g, be, mask, Wn)


def _back_body(acc_ref, u_ref, dinv_ref, b_ref, g_ref, be_ref, mask_ref,
               out_ref):
    z, _ = _post_block(acc_ref, u_ref, dinv_ref, b_ref, g_ref, be_ref,
                       mask_ref)
    out_ref[...] = z


def _tc_back(acc, u, dinv, b, g, be, mask):
    return pl.pallas_call(
        _back_body,
        out_shape=jax.ShapeDtypeStruct((N_PAD, u.shape[1]), jnp.float32),
    )(acc, u, dinv, b, g, be, mask)


def kernel(x, edge_index, W1, b1, g1, be1, W2, b2, g2, be2,
           W3, b3, g3, be3, W4, b4, g4, be4):
    f32 = jnp.float32
    src = edge_index[0]
    dst = edge_index[1]

    npad = E_PAD - E
    pad_idx = (N + (jnp.arange(npad, dtype=jnp.int32) % (N_PAD - N)))
    src_p = jnp.concatenate([src, pad_idx])
    dst_p = jnp.concatenate([dst, pad_idx])
    dst2 = dst_p.reshape(E_PAD // 128, 128)

    x_pad = jnp.zeros((N_PAD, x.shape[1]), f32).at[:N].set(x)
    mask = (jnp.arange(N_PAD) < N).astype(f32).reshape(N_PAD, 1)
    zeros128 = jnp.zeros((N_PAD, 128), f32)
    zeros64 = jnp.zeros((N_PAD, 64), f32)
    zeros32 = jnp.zeros((N_PAD, 32), f32)
    zeros1 = jnp.zeros((N_PAD,), f32)
    ones128 = jnp.ones((128,), f32)

    deg2 = _make_deg_kernel()(dst2, zeros1, ones128)
    deg2t = deg2.T

    agg128 = _make_agg_kernel(128, 128)
    agg64 = _make_agg_kernel(64, 512)
    agg32 = _make_agg_kernel(32, 1024)

    u1, dinv = _tc_front(deg2t, x_pad, W1)
    acc = agg128(u1, src_p, dst2, zeros128)
    u2 = _tc_mid(acc, u1, dinv, b1.reshape(1, -1), g1.reshape(1, -1),
                 be1.reshape(1, -1), mask, W2)
    acc = agg128(u2, src_p, dst2, zeros128)
    u3 = _tc_mid(acc, u2, dinv, b2.reshape(1, -1), g2.reshape(1, -1),
                 be2.reshape(1, -1), mask, W3)
    acc = agg64(u3, src_p, dst2, zeros64)
    u4 = _tc_mid(acc, u3, dinv, b3.reshape(1, -1), g3.reshape(1, -1),
                 be3.reshape(1, -1), mask, W4)
    acc = agg32(u4, src_p, dst2, zeros32)
    z4 = _tc_back(acc, u4, dinv, b4.reshape(1, -1), g4.reshape(1, -1),
                  be4.reshape(1, -1), mask)
    return z4[:N]


# speedup vs baseline: 24.6134x; 1.0074x over previous
import functools

import jax
import jax.numpy as jnp
from jax import lax
from jax.experimental import pallas as pl
from jax.experimental.pallas import tpu as pltpu
from jax.experimental.pallas import tpu_sc as plsc

N = 10000
N_PAD = 10240
E = 320000
E_PAD = 327680
NC = 2
NS = 16
NW = NC * NS
ROWS_PER_TILE = N_PAD // NS
EDGES_PER_WORKER = E_PAD // NW


def _mesh():
    return plsc.VectorSubcoreMesh(
        core_axis_name="c", subcore_axis_name="s", num_cores=NC, num_subcores=NS
    )


def _make_deg_kernel():
    CE = 512
    G = CE // 128
    CHUNKS = EDGES_PER_WORKER // CE

    @functools.partial(
        pl.kernel,
        out_type=jax.ShapeDtypeStruct((NC, N_PAD), jnp.float32),
        mesh=_mesh(),
        scratch_types=[
            pltpu.VMEM((G, 128), jnp.int32),
            pltpu.VMEM((128,), jnp.float32),
            pltpu.VMEM_SHARED((N_PAD,), jnp.float32),
        ],
    )
    def deg_kernel(dst2_hbm, zeros_hbm, ones_hbm, out_hbm, dstv, onesv, acc):
        cid = lax.axis_index("c")
        sid = lax.axis_index("s")
        w = sid * NC + cid
        pltpu.sync_copy(
            zeros_hbm.at[pl.ds(sid * ROWS_PER_TILE, ROWS_PER_TILE)],
            acc.at[pl.ds(sid * ROWS_PER_TILE, ROWS_PER_TILE)],
        )
        pltpu.sync_copy(ones_hbm, onesv)
        plsc.subcore_barrier()

        row_base = w * (EDGES_PER_WORKER // 128)

        def body(g, carry):
            r0 = row_base + g * G
            pltpu.sync_copy(dst2_hbm.at[pl.ds(r0, G)], dstv)
            for j in range(G):
                pltpu.sync_copy(onesv, acc.at[dstv.at[j]], add=True)
            return carry

        lax.fori_loop(0, CHUNKS, body, 0)
        plsc.subcore_barrier()
        pltpu.sync_copy(
            acc.at[pl.ds(sid * ROWS_PER_TILE, ROWS_PER_TILE)],
            out_hbm.at[cid, pl.ds(sid * ROWS_PER_TILE, ROWS_PER_TILE)],
        )

    return deg_kernel


def _make_agg_kernel(D, CE):
    G = CE // 128
    CHUNKS = EDGES_PER_WORKER // CE

    @functools.partial(
        pl.kernel,
        out_type=jax.ShapeDtypeStruct((NC, N_PAD, D), jnp.float32),
        mesh=_mesh(),
        scratch_types=[
            pltpu.VMEM((CE,), jnp.int32),
            pltpu.VMEM((CE,), jnp.int32),
            pltpu.VMEM((G, 128), jnp.int32),
            pltpu.VMEM((G, 128), jnp.int32),
            pltpu.VMEM((CE, D), jnp.float32),
            pltpu.VMEM((CE, D), jnp.float32),
            pltpu.VMEM_SHARED((N_PAD, D), jnp.float32),
            pltpu.SemaphoreType.DMA,
            pltpu.SemaphoreType.DMA,
        ],
        compiler_params=pltpu.CompilerParams(use_tc_tiling_on_sc=False),
    )
    def agg_kernel(u_hbm, src_hbm, dst2_hbm, zeros_hbm, out_hbm,
                   srcA, srcB, dstA, dstB, rowsA, rowsB, acc, semA, semB):
        cid = lax.axis_index("c")
        sid = lax.axis_index("s")
        w = sid * NC + cid

        @pl.when(cid == 0)
        def _():
            pltpu.sync_copy(
                u_hbm.at[pl.ds(sid * ROWS_PER_TILE, ROWS_PER_TILE)],
                acc.at[pl.ds(sid * ROWS_PER_TILE, ROWS_PER_TILE)],
            )

        @pl.when(cid != 0)
        def _():
            pltpu.sync_copy(
                zeros_hbm.at[pl.ds(sid * ROWS_PER_TILE, ROWS_PER_TILE)],
                acc.at[pl.ds(sid * ROWS_PER_TILE, ROWS_PER_TILE)],
            )

        plsc.subcore_barrier()

        base = w * EDGES_PER_WORKER
        row_base = w * (EDGES_PER_WORKER // 128)

        def load_idx(g, srcv, dstv):
            pltpu.sync_copy(src_hbm.at[pl.ds(base + g * CE, CE)], srcv)
            pltpu.sync_copy(dst2_hbm.at[pl.ds(row_base + g * G, G)], dstv)

        def scatter(rowsv, dstv):
            for j in range(G):
                pltpu.sync_copy(
                    rowsv.at[pl.ds(j * 128, 128)],
                    acc.at[dstv.at[j]],
                    add=True,
                )

        load_idx(0, srcA, dstA)
        pltpu.async_copy(u_hbm.at[srcA], rowsA, semA)

        def body(i, carry):
            g0 = 2 * i
            load_idx(g0 + 1, srcB, dstB)
            pltpu.async_copy(u_hbm.at[srcB], rowsB, semB)
            pltpu.make_async_copy(u_hbm.at[srcA], rowsA, semA).wait()
            scatter(rowsA, dstA)

            @pl.when(g0 + 2 < CHUNKS)
            def _():
                load_idx(g0 + 2, srcA, dstA)
                pltpu.async_copy(u_hbm.at[srcA], rowsA, semA)

            pltpu.make_async_copy(u_hbm.at[srcB], rowsB, semB).wait()
            scatter(rowsB, dstB)
            return carry

        lax.fori_loop(0, CHUNKS // 2, body, 0)
        plsc.subcore_barrier()
        pltpu.sync_copy(
            acc.at[pl.ds(sid * ROWS_PER_TILE, ROWS_PER_TILE)],
            out_hbm.at[cid, pl.ds(sid * ROWS_PER_TILE, ROWS_PER_TILE)],
        )

    return agg_kernel


def _matmul_body(x_ref, w_ref, out_ref):
    out_ref[...] = jnp.dot(x_ref[...], w_ref[...],
                           preferred_element_type=jnp.float32)


def _tc_matmul(x_pad, W1):
    return pl.pallas_call(
        _matmul_body,
        out_shape=jax.ShapeDtypeStruct((N_PAD, W1.shape[1]), jnp.float32),
    )(x_pad, W1)


def _front_body(deg2t_ref, p_ref, u_ref, dinv_ref):
    deg = deg2t_ref[:, 0:1] + deg2t_ref[:, 1:2] + 1.0
    dinv = lax.rsqrt(jnp.maximum(deg, 1.0))
    dinv_ref[...] = dinv
    u_ref[...] = p_ref[...] * dinv


def _tc_front(deg2t, p1):
    return pl.pallas_call(
        _front_body,
        out_shape=(
            jax.ShapeDtypeStruct(p1.shape, jnp.float32),
            jax.ShapeDtypeStruct((N_PAD, 1), jnp.float32),
        ),
    )(deg2t, p1)


def _post_block(acc_ref, dinv_ref, b_ref, g_ref, be_ref, mask_ref):
    dinv = dinv_ref[...]
    mask = mask_ref[...]
    conv = (acc_ref[0] + acc_ref[1]) * dinv + b_ref[...]
    t = jnp.maximum(conv, 0.0) * mask
    m = jnp.sum(t, axis=0, keepdims=True) * (1.0 / N)
    d = (t - m) * mask
    v = jnp.sum(d * d, axis=0, keepdims=True) * (1.0 / N)
    z = (g_ref[...] * d * lax.rsqrt(v + 1e-5) + be_ref[...]) * mask
    return z, dinv


def _mid_body(acc_ref, dinv_ref, b_ref, g_ref, be_ref, mask_ref,
              w_ref, out_ref):
    z, dinv = _post_block(acc_ref, dinv_ref, b_ref, g_ref, be_ref, mask_ref)
    out_ref[...] = jnp.dot(z, w_ref[...],
                           preferred_element_type=jnp.float32) * dinv


def _tc_mid(acc, dinv, b, g, be, mask, Wn):
    return pl.pallas_call(
        _mid_body,
        out_shape=jax.ShapeDtypeStruct((N_PAD, Wn.shape[1]), jnp.float32),
    )(acc, dinv, b, g, be, mask, Wn)


def _back_body(acc_ref, dinv_ref, b_ref, g_ref, be_ref, mask_ref, out_ref):
    z, _ = _post_block(acc_ref, dinv_ref, b_ref, g_ref, be_ref, mask_ref)
    out_ref[...] = z


def _tc_back(acc, dinv, b, g, be, mask):
    return pl.pallas_call(
        _back_body,
        out_shape=jax.ShapeDtypeStruct((N_PAD, acc.shape[2]), jnp.float32),
    )(acc, dinv, b, g, be, mask)


def kernel(x, edge_index, W1, b1, g1, be1, W2, b2, g2, be2,
           W3, b3, g3, be3, W4, b4, g4, be4):
    f32 = jnp.float32
    src = edge_index[0]
    dst = edge_index[1]

    npad = E_PAD - E
    pad_idx = (N + (jnp.arange(npad, dtype=jnp.int32) % (N_PAD - N)))
    src_p = jnp.concatenate([src, pad_idx])
    dst_p = jnp.concatenate([dst, pad_idx])
    dst2 = dst_p.reshape(E_PAD // 128, 128)

    x_pad = jnp.zeros((N_PAD, x.shape[1]), f32).at[:N].set(x)
    mask = (jnp.arange(N_PAD) < N).astype(f32).reshape(N_PAD, 1)
    zeros128 = jnp.zeros((N_PAD, 128), f32)
    zeros64 = jnp.zeros((N_PAD, 64), f32)
    zeros32 = jnp.zeros((N_PAD, 32), f32)
    zeros1 = jnp.zeros((N_PAD,), f32)
    ones128 = jnp.ones((128,), f32)

    deg2 = _make_deg_kernel()(dst2, zeros1, ones128)
    deg2t = deg2.T

    agg128 = _make_agg_kernel(128, 128)
    agg64 = _make_agg_kernel(64, 512)
    agg32 = _make_agg_kernel(32, 1024)

    p1 = _tc_matmul(x_pad, W1)
    u1, dinv = _tc_front(deg2t, p1)
    acc = agg128(u1, src_p, dst2, zeros128)
    u2 = _tc_mid(acc, dinv, b1.reshape(1, -1), g1.reshape(1, -1),
                 be1.reshape(1, -1), mask, W2)
    acc = agg128(u2, src_p, dst2, zeros128)
    u3 = _tc_mid(acc, dinv, b2.reshape(1, -1), g2.reshape(1, -1),
                 be2.reshape(1, -1), mask, W3)
    acc = agg64(u3, src_p, dst2, zeros64)
    u4 = _tc_mid(acc, dinv, b3.reshape(1, -1), g3.reshape(1, -1),
                 be3.reshape(1, -1), mask, W4)
    acc = agg32(u4, src_p, dst2, zeros32)
    z4 = _tc_back(acc, dinv, b4.reshape(1, -1), g4.reshape(1, -1),
                  be4.reshape(1, -1), mask)
    return z4[:N]


# speedup vs baseline: 27.0704x; 1.0998x over previous
import functools

import jax
import jax.numpy as jnp
from jax import lax
from jax.experimental import pallas as pl
from jax.experimental.pallas import tpu as pltpu
from jax.experimental.pallas import tpu_sc as plsc

N = 10000
N_PAD = 10240
E = 320000
E_PAD = 327680
NC = 2
NS = 16
NW = NC * NS
ROWS_PER_TILE = N_PAD // NS
EDGES_PER_WORKER = E_PAD // NW


def _mesh():
    return plsc.VectorSubcoreMesh(
        core_axis_name="c", subcore_axis_name="s", num_cores=NC, num_subcores=NS
    )


def _make_deg_kernel():
    CE = 512
    G = CE // 128
    CHUNKS = EDGES_PER_WORKER // CE

    @functools.partial(
        pl.kernel,
        out_type=jax.ShapeDtypeStruct((NC, N_PAD), jnp.float32),
        mesh=_mesh(),
        scratch_types=[
            pltpu.VMEM((G, 128), jnp.int32),
            pltpu.VMEM((128,), jnp.float32),
            pltpu.VMEM_SHARED((N_PAD,), jnp.float32),
        ],
    )
    def deg_kernel(dst2_hbm, zeros_hbm, ones_hbm, out_hbm, dstv, onesv, acc):
        cid = lax.axis_index("c")
        sid = lax.axis_index("s")
        w = sid * NC + cid
        pltpu.sync_copy(
            zeros_hbm.at[pl.ds(sid * ROWS_PER_TILE, ROWS_PER_TILE)],
            acc.at[pl.ds(sid * ROWS_PER_TILE, ROWS_PER_TILE)],
        )
        pltpu.sync_copy(ones_hbm, onesv)
        plsc.subcore_barrier()

        row_base = w * (EDGES_PER_WORKER // 128)

        def body(g, carry):
            r0 = row_base + g * G
            pltpu.sync_copy(dst2_hbm.at[pl.ds(r0, G)], dstv)
            for j in range(G):
                pltpu.sync_copy(onesv, acc.at[dstv.at[j]], add=True)
            return carry

        lax.fori_loop(0, CHUNKS, body, 0)
        plsc.subcore_barrier()
        pltpu.sync_copy(
            acc.at[pl.ds(sid * ROWS_PER_TILE, ROWS_PER_TILE)],
            out_hbm.at[cid, pl.ds(sid * ROWS_PER_TILE, ROWS_PER_TILE)],
        )

    return deg_kernel


def _make_agg_kernel_packed(D):
    CE = 128
    CHUNKS = EDGES_PER_WORKER // CE

    @functools.partial(
        pl.kernel,
        out_type=jax.ShapeDtypeStruct((NC, N_PAD, D), jnp.float32),
        mesh=_mesh(),
        scratch_types=[
            pltpu.VMEM((1, 2, 128), jnp.int32),
            pltpu.VMEM((1, 2, 128), jnp.int32),
            pltpu.VMEM((CE, D), jnp.float32),
            pltpu.VMEM((CE, D), jnp.float32),
            pltpu.VMEM_SHARED((N_PAD, D), jnp.float32),
            pltpu.SemaphoreType.DMA,
            pltpu.SemaphoreType.DMA,
        ],
        compiler_params=pltpu.CompilerParams(use_tc_tiling_on_sc=False),
    )
    def agg_kernel(u_hbm, idx2_hbm, zeros_hbm, out_hbm,
                   idxA, idxB, rowsA, rowsB, acc, semA, semB):
        cid = lax.axis_index("c")
        sid = lax.axis_index("s")
        w = sid * NC + cid

        @pl.when(cid == 0)
        def _():
            pltpu.sync_copy(
                u_hbm.at[pl.ds(sid * ROWS_PER_TILE, ROWS_PER_TILE)],
                acc.at[pl.ds(sid * ROWS_PER_TILE, ROWS_PER_TILE)],
            )

        @pl.when(cid != 0)
        def _():
            pltpu.sync_copy(
                zeros_hbm.at[pl.ds(sid * ROWS_PER_TILE, ROWS_PER_TILE)],
                acc.at[pl.ds(sid * ROWS_PER_TILE, ROWS_PER_TILE)],
            )

        plsc.subcore_barrier()

        row_base = w * CHUNKS

        def load_idx(g, idxv):
            pltpu.sync_copy(idx2_hbm.at[pl.ds(row_base + g, 1)], idxv)

        def scatter(rowsv, idxv):
            pltpu.sync_copy(rowsv, acc.at[idxv.at[0, 1]], add=True)

        load_idx(0, idxA)
        pltpu.async_copy(u_hbm.at[idxA.at[0, 0]], rowsA, semA)

        def body(i, carry):
            g0 = 2 * i
            load_idx(g0 + 1, idxB)
            pltpu.async_copy(u_hbm.at[idxB.at[0, 0]], rowsB, semB)
            pltpu.make_async_copy(u_hbm.at[idxA.at[0, 0]], rowsA, semA).wait()
            scatter(rowsA, idxA)

            @pl.when(g0 + 2 < CHUNKS)
            def _():
                load_idx(g0 + 2, idxA)
                pltpu.async_copy(u_hbm.at[idxA.at[0, 0]], rowsA, semA)

            pltpu.make_async_copy(u_hbm.at[idxB.at[0, 0]], rowsB, semB).wait()
            scatter(rowsB, idxB)
            return carry

        lax.fori_loop(0, CHUNKS // 2, body, 0)
        plsc.subcore_barrier()
        pltpu.sync_copy(
            acc.at[pl.ds(sid * ROWS_PER_TILE, ROWS_PER_TILE)],
            out_hbm.at[cid, pl.ds(sid * ROWS_PER_TILE, ROWS_PER_TILE)],
        )

    return agg_kernel


def _make_agg_kernel(D, CE):
    G = CE // 128
    CHUNKS = EDGES_PER_WORKER // CE

    @functools.partial(
        pl.kernel,
        out_type=jax.ShapeDtypeStruct((NC, N_PAD, D), jnp.float32),
        mesh=_mesh(),
        scratch_types=[
            pltpu.VMEM((CE,), jnp.int32),
            pltpu.VMEM((CE,), jnp.int32),
            pltpu.VMEM((G, 128), jnp.int32),
            pltpu.VMEM((G, 128), jnp.int32),
            pltpu.VMEM((CE, D), jnp.float32),
            pltpu.VMEM((CE, D), jnp.float32),
            pltpu.VMEM_SHARED((N_PAD, D), jnp.float32),
            pltpu.SemaphoreType.DMA,
            pltpu.SemaphoreType.DMA,
        ],
        compiler_params=pltpu.CompilerParams(use_tc_tiling_on_sc=False),
    )
    def agg_kernel(u_hbm, src_hbm, dst2_hbm, zeros_hbm, out_hbm,
                   srcA, srcB, dstA, dstB, rowsA, rowsB, acc, semA, semB):
        cid = lax.axis_index("c")
        sid = lax.axis_index("s")
        w = sid * NC + cid

        @pl.when(cid == 0)
        def _():
            pltpu.sync_copy(
                u_hbm.at[pl.ds(sid * ROWS_PER_TILE, ROWS_PER_TILE)],
                acc.at[pl.ds(sid * ROWS_PER_TILE, ROWS_PER_TILE)],
            )

        @pl.when(cid != 0)
        def _():
            pltpu.sync_copy(
                zeros_hbm.at[pl.ds(sid * ROWS_PER_TILE, ROWS_PER_TILE)],
                acc.at[pl.ds(sid * ROWS_PER_TILE, ROWS_PER_TILE)],
            )

        plsc.subcore_barrier()

        base = w * EDGES_PER_WORKER
        row_base = w * (EDGES_PER_WORKER // 128)

        def load_idx(g, srcv, dstv):
            pltpu.sync_copy(src_hbm.at[pl.ds(base + g * CE, CE)], srcv)
            pltpu.sync_copy(dst2_hbm.at[pl.ds(row_base + g * G, G)], dstv)

        def scatter(rowsv, dstv):
            for j in range(G):
                pltpu.sync_copy(
                    rowsv.at[pl.ds(j * 128, 128)],
                    acc.at[dstv.at[j]],
                    add=True,
                )

        load_idx(0, srcA, dstA)
        pltpu.async_copy(u_hbm.at[srcA], rowsA, semA)

        def body(i, carry):
            g0 = 2 * i
            load_idx(g0 + 1, srcB, dstB)
            pltpu.async_copy(u_hbm.at[srcB], rowsB, semB)
            pltpu.make_async_copy(u_hbm.at[srcA], rowsA, semA).wait()
            scatter(rowsA, dstA)

            @pl.when(g0 + 2 < CHUNKS)
            def _():
                load_idx(g0 + 2, srcA, dstA)
                pltpu.async_copy(u_hbm.at[srcA], rowsA, semA)

            pltpu.make_async_copy(u_hbm.at[srcB], rowsB, semB).wait()
            scatter(rowsB, dstB)
            return carry

        lax.fori_loop(0, CHUNKS // 2, body, 0)
        plsc.subcore_barrier()
        pltpu.sync_copy(
            acc.at[pl.ds(sid * ROWS_PER_TILE, ROWS_PER_TILE)],
            out_hbm.at[cid, pl.ds(sid * ROWS_PER_TILE, ROWS_PER_TILE)],
        )

    return agg_kernel


def _matmul_body(x_ref, w_ref, out_ref):
    out_ref[...] = jnp.dot(x_ref[...], w_ref[...],
                           preferred_element_type=jnp.float32)


def _tc_matmul(x_pad, W1):
    return pl.pallas_call(
        _matmul_body,
        out_shape=jax.ShapeDtypeStruct((N_PAD, W1.shape[1]), jnp.float32),
    )(x_pad, W1)


def _front_body(deg2t_ref, p_ref, u_ref, dinv_ref):
    deg = deg2t_ref[:, 0:1] + deg2t_ref[:, 1:2] + 1.0
    dinv = lax.rsqrt(jnp.maximum(deg, 1.0))
    dinv_ref[...] = dinv
    u_ref[...] = p_ref[...] * dinv


def _tc_front(deg2t, p1):
    return pl.pallas_call(
        _front_body,
        out_shape=(
            jax.ShapeDtypeStruct(p1.shape, jnp.float32),
            jax.ShapeDtypeStruct((N_PAD, 1), jnp.float32),
        ),
    )(deg2t, p1)


def _post_block(acc_ref, dinv_ref, b_ref, g_ref, be_ref, mask_ref):
    dinv = dinv_ref[...]
    mask = mask_ref[...]
    conv = (acc_ref[0] + acc_ref[1]) * dinv + b_ref[...]
    t = jnp.maximum(conv, 0.0) * mask
    m = jnp.sum(t, axis=0, keepdims=True) * (1.0 / N)
    d = (t - m) * mask
    v = jnp.sum(d * d, axis=0, keepdims=True) * (1.0 / N)
    z = (g_ref[...] * d * lax.rsqrt(v + 1e-5) + be_ref[...]) * mask
    return z, dinv


def _mid_body(acc_ref, dinv_ref, b_ref, g_ref, be_ref, mask_ref,
              w_ref, out_ref):
    z, dinv = _post_block(acc_ref, dinv_ref, b_ref, g_ref, be_ref, mask_ref)
    out_ref[...] = jnp.dot(z, w_ref[...],
                           preferred_element_type=jnp.float32) * dinv


def _tc_mid(acc, dinv, b, g, be, mask, Wn):
    return pl.pallas_call(
        _mid_body,
        out_shape=jax.ShapeDtypeStruct((N_PAD, Wn.shape[1]), jnp.float32),
    )(acc, dinv, b, g, be, mask, Wn)


def _back_body(acc_ref, dinv_ref, b_ref, g_ref, be_ref, mask_ref, out_ref):
    z, _ = _post_block(acc_ref, dinv_ref, b_ref, g_ref, be_ref, mask_ref)
    out_ref[...] = z


def _tc_back(acc, dinv, b, g, be, mask):
    return pl.pallas_call(
        _back_body,
        out_shape=jax.ShapeDtypeStruct((N_PAD, acc.shape[2]), jnp.float32),
    )(acc, dinv, b, g, be, mask)


def kernel(x, edge_index, W1, b1, g1, be1, W2, b2, g2, be2,
           W3, b3, g3, be3, W4, b4, g4, be4):
    f32 = jnp.float32
    src = edge_index[0]
    dst = edge_index[1]

    npad = E_PAD - E
    pad_idx = (N + (jnp.arange(npad, dtype=jnp.int32) % (N_PAD - N)))
    src_p = jnp.concatenate([src, pad_idx])
    dst_p = jnp.concatenate([dst, pad_idx])
    dst2 = dst_p.reshape(E_PAD // 128, 128)
    idx2 = jnp.stack([src_p.reshape(E_PAD // 128, 128), dst2], axis=1)

    x_pad = jnp.zeros((N_PAD, x.shape[1]), f32).at[:N].set(x)
    mask = (jnp.arange(N_PAD) < N).astype(f32).reshape(N_PAD, 1)
    zeros128 = jnp.zeros((N_PAD, 128), f32)
    zeros64 = jnp.zeros((N_PAD, 64), f32)
    zeros32 = jnp.zeros((N_PAD, 32), f32)
    zeros1 = jnp.zeros((N_PAD,), f32)
    ones128 = jnp.ones((128,), f32)

    deg2 = _make_deg_kernel()(dst2, zeros1, ones128)
    deg2t = deg2.T

    agg128 = _make_agg_kernel_packed(128)
    agg64 = _make_agg_kernel(64, 512)
    agg32 = _make_agg_kernel(32, 1024)

    p1 = _tc_matmul(x_pad, W1)
    u1, dinv = _tc_front(deg2t, p1)
    acc = agg128(u1, idx2, zeros128)
    u2 = _tc_mid(acc, dinv, b1.reshape(1, -1), g1.reshape(1, -1),
                 be1.reshape(1, -1), mask, W2)
    acc = agg128(u2, idx2, zeros128)
    u3 = _tc_mid(acc, dinv, b2.reshape(1, -1), g2.reshape(1, -1),
                 be2.reshape(1, -1), mask, W3)
    acc = agg64(u3, src_p, dst2, zeros64)
    u4 = _tc_mid(acc, dinv, b3.reshape(1, -1), g3.reshape(1, -1),
                 be3.reshape(1, -1), mask, W4)
    acc = agg32(u4, src_p, dst2, zeros32)
    z4 = _tc_back(acc, dinv, b4.reshape(1, -1), g4.reshape(1, -1),
                  be4.reshape(1, -1), mask)
    return z4[:N]
